# baseline (speedup 1.0000x reference)
import functools

import jax
import jax.numpy as jnp
import numpy as np
from jax import lax
from jax.experimental import pallas as pl
from jax.experimental.pallas import tpu as pltpu
from jax.experimental.pallas import tpu_sc as plsc

NENTITY = 100000
NRELATION = 500
DIM = 128
GAMMA = np.float32(24.0)
ALPHA = np.float32(0.02)
BATCH = 1024
NNEG = 256

NC = 2
NS = 16
NW = NC * NS
BPW = BATCH // NW
L = 16
KCH = DIM // L
GROW = 128
NEGROWS = BATCH * NNEG // GROW
GPW = NEGROWS // NW
C1 = np.float32(1.0 - 0.02)


def _sl(k):
    return pl.ds(k * L, L)


def _sc_body(ent_hbm, off_hbm, ans_hbm, cm_hbm, ca_hbm, om_hbm, oa_hbm,
             anch_hbm, rel_hbm, posi_hbm, neg_hbm,
             pos_out, neg_out,
             idx_a, idx_r, idx_p, negidx,
             bufA, bufB, bufC, bufD, bufE, bufF, bufG,
             centers, offsets, posrow, nbuf, nbuf2, nbuf3, nbuf4,
             negloc, tbuf, tbuf2, semA, semB, semC, semD, semA2):
    wid = lax.axis_index("s") * NC + lax.axis_index("c")
    base = wid * BPW
    nrings = (nbuf, nbuf2, nbuf3, nbuf4)
    nsems = (semA, semB, semC, semD)

    pltpu.sync_copy(anch_hbm.at[pl.ds(base, BPW)], idx_a)
    pltpu.sync_copy(rel_hbm.at[pl.ds(base, BPW)], idx_r)
    pltpu.sync_copy(posi_hbm.at[pl.ds(base, BPW)], idx_p)
    pltpu.sync_copy(neg_hbm.at[pl.ds(wid * GPW, GPW)], negidx)

    pltpu.async_copy(ent_hbm.at[idx_a], bufA, semA2)
    pltpu.async_copy(cm_hbm.at[idx_r], bufB, semA2)
    pltpu.async_copy(ca_hbm.at[idx_r], bufC, semA2)
    pltpu.async_copy(off_hbm.at[idx_a], bufD, semA2)
    pltpu.async_copy(om_hbm.at[idx_r], bufE, semA2)
    pltpu.async_copy(oa_hbm.at[idx_r], bufF, semA2)
    pltpu.async_copy(ans_hbm.at[idx_p], bufG, semA2)
    for r in range(4):
        pltpu.async_copy(ans_hbm.at[negidx.at[r]], nrings[r], nsems[r])
    for dst in (bufA, bufB, bufC, bufD, bufE, bufF, bufG):
        pltpu.make_async_copy(ent_hbm.at[idx_a], dst, semA2).wait()

    lanes = lax.iota(jnp.int32, L)
    colbase = lanes * L

    def _tree(vs):
        while len(vs) > 1:
            pair = [a + b for a, b in zip(vs[::2], vs[1::2])]
            if len(vs) % 2:
                pair.append(vs[-1])
            vs = pair
        return vs[0]

    def col_totals(tb=None):
        tb = tbuf if tb is None else tb
        return _tree([plsc.load_gather(tb, [colbase + k]) for k in range(L)])

    def bf_abs(d):
        u = plsc.bitcast(d, jnp.int32)
        return plsc.bitcast(u & jnp.int32(0x7FFF7FFF), jnp.bfloat16)

    def bf_split_sum(s):
        u = plsc.bitcast(s, jnp.int32)
        hi = plsc.bitcast(u & jnp.int32(-65536), jnp.float32)
        lo = plsc.bitcast(lax.shift_left(u, 16), jnp.float32)
        return hi + lo

    def b_body(grp, carry):
        for i in range(L):
            b = grp * L + i
            ds, ms = [], []
            for k in range(KCH):
                c = bufA[b, _sl(k)] * bufB[b, _sl(k)] + bufC[b, _sl(k)]
                o = jnp.maximum(
                    bufD[b, _sl(k)] * bufE[b, _sl(k)] + bufF[b, _sl(k)],
                    np.float32(0.0))
                centers[b, _sl(k)] = c
                offsets[b, _sl(k)] = o
                d = jnp.abs(bufG[b, _sl(k)] - c)
                ds.append(d)
                ms.append(jnp.minimum(d, o))
            tbuf[pl.ds(i * L, L)] = _tree(ds) - C1 * _tree(ms)
        posrow[pl.ds(grp * L, L)] = GAMMA - col_totals()
        return carry

    lax.fori_loop(0, BPW // L, b_body, 0)
    pltpu.sync_copy(posrow, pos_out.at[pl.ds(base, BPW)])

    def pack_box(b):
        cvp = [plsc.pack(centers[b, _sl(2 * k)], centers[b, _sl(2 * k + 1)], format=plsc.PackFormat.INTERLEAVED)
               for k in range(KCH // 2)]
        ovp = [plsc.pack(offsets[b, _sl(2 * k)], offsets[b, _sl(2 * k + 1)], format=plsc.PackFormat.INTERLEAVED)
               for k in range(KCH // 2)]
        return cvp, ovp

    def compute_group(g, buf, cvp, ovp):
        def j_body(jg, jcarry):
            ts = []
            for i in range(L):
                j = jg * L + i
                ds, ms = [], []
                for k in range(KCH // 2):
                    ap = plsc.pack(buf[j, _sl(2 * k)], buf[j, _sl(2 * k + 1)], format=plsc.PackFormat.INTERLEAVED)
                    d = bf_abs(ap - cvp[k])
                    ds.append(d)
                    ms.append(jnp.minimum(d, ovp[k]))
                ts.append(bf_split_sum(_tree(ds))
                          - C1 * bf_split_sum(_tree(ms)))
            for i in range(L):
                tbuf[pl.ds(i * L, L)] = ts[i]
            negloc[g, pl.ds(jg * L, L)] = GAMMA - col_totals()
            return jcarry

        lax.fori_loop(0, GROW // L, j_body, 0, unroll=2)

    def g_body(i, carry):
        g0 = i * 4
        for r in range(4):
            g = g0 + r
            if r % 2 == 0:
                cvp, ovp = pack_box(i * 2 + r // 2)
            pltpu.make_async_copy(
                ans_hbm.at[negidx.at[g]], nrings[r], nsems[r]).wait()
            compute_group(g, nrings[r], cvp, ovp)

            @pl.when(i < GPW // 4 - 1)
            def _():
                pltpu.async_copy(ans_hbm.at[negidx.at[g + 4]], nrings[r], nsems[r])

        return carry

    lax.fori_loop(0, GPW // 4, g_body, 0)
    pltpu.sync_copy(negloc, neg_out.at[pl.ds(wid * GPW, GPW)])


_sc_logits = functools.partial(
    pl.kernel,
    out_type=(jax.ShapeDtypeStruct((BATCH,), jnp.float32),
              jax.ShapeDtypeStruct((NEGROWS, GROW), jnp.float32)),
    mesh=plsc.VectorSubcoreMesh(core_axis_name="c", subcore_axis_name="s"),
    compiler_params=pltpu.CompilerParams(needs_layout_passes=False),
    scratch_types=(
        pltpu.VMEM((BPW,), jnp.int32),
        pltpu.VMEM((BPW,), jnp.int32),
        pltpu.VMEM((BPW,), jnp.int32),
        pltpu.VMEM((GPW, GROW), jnp.int32),
        pltpu.VMEM((BPW, DIM), jnp.float32),
        pltpu.VMEM((BPW, DIM), jnp.float32),
        pltpu.VMEM((BPW, DIM), jnp.float32),
        pltpu.VMEM((BPW, DIM), jnp.float32),
        pltpu.VMEM((BPW, DIM), jnp.float32),
        pltpu.VMEM((BPW, DIM), jnp.float32),
        pltpu.VMEM((BPW, DIM), jnp.float32),
        pltpu.VMEM((BPW, DIM), jnp.float32),
        pltpu.VMEM((BPW, DIM), jnp.float32),
        pltpu.VMEM((BPW,), jnp.float32),
        pltpu.VMEM((GROW, DIM), jnp.float32),
        pltpu.VMEM((GROW, DIM), jnp.float32),
        pltpu.VMEM((GROW, DIM), jnp.float32),
        pltpu.VMEM((GROW, DIM), jnp.float32),
        pltpu.VMEM((GPW, GROW), jnp.float32),
        pltpu.VMEM((L * L,), jnp.float32),
        pltpu.VMEM((L * L,), jnp.float32),
        pltpu.SemaphoreType.DMA,
        pltpu.SemaphoreType.DMA,
        pltpu.SemaphoreType.DMA,
        pltpu.SemaphoreType.DMA,
        pltpu.SemaphoreType.DMA,
    ),
)(_sc_body)


def _loss_body(pos_ref, neg_ref, w8_ref, wc_ref, out_ref):
    def logsig(x):
        return jnp.minimum(x, np.float32(0.0)) - jnp.log1p(jnp.exp(-jnp.abs(x)))

    w8 = w8_ref[...]
    sum_w = jnp.sum(w8)
    pos_term = jnp.sum(w8 * logsig(pos_ref[...]))
    neg_term = jnp.sum(wc_ref[...] * logsig(-neg_ref[...])) / np.float32(NNEG)
    out_ref[0, 0] = -(pos_term + neg_term) / (np.float32(2.0) * sum_w)


_loss_tc = pl.pallas_call(
    _loss_body,
    out_shape=jax.ShapeDtypeStruct((1, 1), jnp.float32),
    out_specs=pl.BlockSpec(memory_space=pltpu.SMEM),
)


def kernel(entity_embedding, offset_embedding, answer_embedding,
           center_mul, center_add, offset_mul, offset_add,
           subsampling_weight, anchors, relations,
           positive_sample, negative_sample):
    anchors = anchors.astype(jnp.int32)
    relations = relations.astype(jnp.int32)
    positive_sample = positive_sample.astype(jnp.int32)
    neg2 = negative_sample.astype(jnp.int32).reshape(NEGROWS, GROW)

    pos_logit, neg_logit = _sc_logits(
        entity_embedding, offset_embedding, answer_embedding,
        center_mul, center_add, offset_mul, offset_add,
        anchors, relations, positive_sample, neg2)

    loss = _loss_tc(pos_logit.reshape(8, DIM),
                    neg_logit.reshape(BATCH, NNEG),
                    subsampling_weight.reshape(8, DIM),
                    subsampling_weight.reshape(BATCH, 1))
    return loss[0, 0]


# speedup vs baseline: 1.1556x; 1.1556x over previous
import functools

import jax
import jax.numpy as jnp
import numpy as np
from jax import lax
from jax.experimental import pallas as pl
from jax.experimental.pallas import tpu as pltpu
from jax.experimental.pallas import tpu_sc as plsc

NENTITY = 100000
NRELATION = 500
DIM = 128
GAMMA = np.float32(24.0)
ALPHA = np.float32(0.02)
BATCH = 1024
NNEG = 256

NC = 2
NS = 16
NW = NC * NS
BPW = BATCH // NW
L = 16
KCH = DIM // L
GROW = 128
NEGROWS = BATCH * NNEG // GROW
GPW = NEGROWS // NW
C1 = np.float32(1.0 - 0.02)


def _sl(k):
    return pl.ds(k * L, L)


def _sc_body(ent_hbm, off_hbm, ans_hbm, cm_hbm, ca_hbm, om_hbm, oa_hbm,
             anch_hbm, rel_hbm, posi_hbm, neg_hbm,
             pos_out, neg_out,
             idx_a, idx_r, idx_p, negidx,
             bufA, bufB, bufC, bufD, bufE, bufF, bufG,
             centers, offsets, posrow, nbuf, nbuf2, nbuf3, nbuf4,
             negloc, tbuf, tbuf2, semA, semB, semC, semD, semA2):
    wid = lax.axis_index("s") * NC + lax.axis_index("c")
    base = wid * BPW
    nrings = (nbuf, nbuf2, nbuf3, nbuf4)
    nsems = (semA, semB, semC, semD)

    pltpu.sync_copy(anch_hbm.at[pl.ds(base, BPW)], idx_a)
    pltpu.sync_copy(rel_hbm.at[pl.ds(base, BPW)], idx_r)
    pltpu.sync_copy(posi_hbm.at[pl.ds(base, BPW)], idx_p)
    pltpu.sync_copy(neg_hbm.at[pl.ds(wid * GPW, GPW)], negidx)

    pltpu.async_copy(ent_hbm.at[idx_a], bufA, semA2)
    pltpu.async_copy(cm_hbm.at[idx_r], bufB, semA2)
    pltpu.async_copy(ca_hbm.at[idx_r], bufC, semA2)
    pltpu.async_copy(off_hbm.at[idx_a], bufD, semA2)
    pltpu.async_copy(om_hbm.at[idx_r], bufE, semA2)
    pltpu.async_copy(oa_hbm.at[idx_r], bufF, semA2)
    pltpu.async_copy(ans_hbm.at[idx_p], bufG, semA2)
    for r in range(4):
        pltpu.async_copy(ans_hbm.at[negidx.at[r]], nrings[r], nsems[r])
    for dst in (bufA, bufB, bufC, bufD, bufE, bufF, bufG):
        pltpu.make_async_copy(ent_hbm.at[idx_a], dst, semA2).wait()

    lanes = lax.iota(jnp.int32, L)
    colbase = lanes * L

    def _tree(vs):
        while len(vs) > 1:
            pair = [a + b for a, b in zip(vs[::2], vs[1::2])]
            if len(vs) % 2:
                pair.append(vs[-1])
            vs = pair
        return vs[0]

    def col_totals(tb=None):
        tb = tbuf if tb is None else tb
        return _tree([plsc.load_gather(tb, [colbase + k]) for k in range(L)])

    def bf_abs(d):
        u = plsc.bitcast(d, jnp.int32)
        return plsc.bitcast(u & jnp.int32(0x7FFF7FFF), jnp.bfloat16)

    def bf_split_sum(s):
        u = plsc.bitcast(s, jnp.int32)
        hi = plsc.bitcast(u & jnp.int32(-65536), jnp.float32)
        lo = plsc.bitcast(lax.shift_left(u, 16), jnp.float32)
        return hi + lo

    def b_body(grp, carry):
        for i in range(L):
            b = grp * L + i
            ds, ms = [], []
            for k in range(KCH):
                c = bufA[b, _sl(k)] * bufB[b, _sl(k)] + bufC[b, _sl(k)]
                o = jnp.maximum(
                    bufD[b, _sl(k)] * bufE[b, _sl(k)] + bufF[b, _sl(k)],
                    np.float32(0.0))
                centers[b, _sl(k)] = c
                offsets[b, _sl(k)] = o
                d = jnp.abs(bufG[b, _sl(k)] - c)
                ds.append(d)
                ms.append(jnp.minimum(d, o))
            tbuf[pl.ds(i * L, L)] = _tree(ds) - C1 * _tree(ms)
        posrow[pl.ds(grp * L, L)] = GAMMA - col_totals()
        return carry

    lax.fori_loop(0, BPW // L, b_body, 0)
    pltpu.sync_copy(posrow, pos_out.at[pl.ds(base, BPW)])

    def pack_box(b):
        cvp = [plsc.pack(centers[b, _sl(2 * k)], centers[b, _sl(2 * k + 1)], format=plsc.PackFormat.INTERLEAVED)
               for k in range(KCH // 2)]
        ovp = [plsc.pack(offsets[b, _sl(2 * k)], offsets[b, _sl(2 * k + 1)], format=plsc.PackFormat.INTERLEAVED)
               for k in range(KCH // 2)]
        return cvp, ovp

    def compute_group(g, buf, cvp, ovp):
        def j_body(jg, jcarry):
            ts = []
            for i in range(L):
                j = jg * L + i
                ds, ms = [], []
                for k in range(KCH // 2):
                    ap = plsc.pack(buf[j, _sl(2 * k)], buf[j, _sl(2 * k + 1)], format=plsc.PackFormat.INTERLEAVED)
                    d = bf_abs(ap - cvp[k])
                    ds.append(d)
                    ms.append(jnp.minimum(d, ovp[k]))
                tb16 = _tree(ds) - jnp.bfloat16(C1) * _tree(ms)
                ts.append(bf_split_sum(tb16))
            for i in range(L):
                tbuf[pl.ds(i * L, L)] = ts[i]
            negloc[g, pl.ds(jg * L, L)] = GAMMA - col_totals()
            return jcarry

        lax.fori_loop(0, GROW // L, j_body, 0)

    def g_body(i, carry):
        g0 = i * 4
        for r in range(4):
            g = g0 + r
            if r % 2 == 0:
                cvp, ovp = pack_box(i * 2 + r // 2)
            pltpu.make_async_copy(
                ans_hbm.at[negidx.at[g]], nrings[r], nsems[r]).wait()
            compute_group(g, nrings[r], cvp, ovp)

            @pl.when(i < GPW // 4 - 1)
            def _():
                pltpu.async_copy(ans_hbm.at[negidx.at[g + 4]], nrings[r], nsems[r])

        return carry

    lax.fori_loop(0, GPW // 4, g_body, 0)
    pltpu.sync_copy(negloc, neg_out.at[pl.ds(wid * GPW, GPW)])


_sc_logits = functools.partial(
    pl.kernel,
    out_type=(jax.ShapeDtypeStruct((BATCH,), jnp.float32),
              jax.ShapeDtypeStruct((NEGROWS, GROW), jnp.float32)),
    mesh=plsc.VectorSubcoreMesh(core_axis_name="c", subcore_axis_name="s"),
    compiler_params=pltpu.CompilerParams(needs_layout_passes=False),
    scratch_types=(
        pltpu.VMEM((BPW,), jnp.int32),
        pltpu.VMEM((BPW,), jnp.int32),
        pltpu.VMEM((BPW,), jnp.int32),
        pltpu.VMEM((GPW, GROW), jnp.int32),
        pltpu.VMEM((BPW, DIM), jnp.float32),
        pltpu.VMEM((BPW, DIM), jnp.float32),
        pltpu.VMEM((BPW, DIM), jnp.float32),
        pltpu.VMEM((BPW, DIM), jnp.float32),
        pltpu.VMEM((BPW, DIM), jnp.float32),
        pltpu.VMEM((BPW, DIM), jnp.float32),
        pltpu.VMEM((BPW, DIM), jnp.float32),
        pltpu.VMEM((BPW, DIM), jnp.float32),
        pltpu.VMEM((BPW, DIM), jnp.float32),
        pltpu.VMEM((BPW,), jnp.float32),
        pltpu.VMEM((GROW, DIM), jnp.float32),
        pltpu.VMEM((GROW, DIM), jnp.float32),
        pltpu.VMEM((GROW, DIM), jnp.float32),
        pltpu.VMEM((GROW, DIM), jnp.float32),
        pltpu.VMEM((GPW, GROW), jnp.float32),
        pltpu.VMEM((L * L,), jnp.float32),
        pltpu.VMEM((L * L,), jnp.float32),
        pltpu.SemaphoreType.DMA,
        pltpu.SemaphoreType.DMA,
        pltpu.SemaphoreType.DMA,
        pltpu.SemaphoreType.DMA,
        pltpu.SemaphoreType.DMA,
    ),
)(_sc_body)


def _loss_body(pos_ref, neg_ref, w8_ref, wc_ref, out_ref):
    def logsig(x):
        return jnp.minimum(x, np.float32(0.0)) - jnp.log1p(jnp.exp(-jnp.abs(x)))

    w8 = w8_ref[...]
    sum_w = jnp.sum(w8)
    pos_term = jnp.sum(w8 * logsig(pos_ref[...]))
    neg_term = jnp.sum(wc_ref[...] * logsig(-neg_ref[...])) / np.float32(NNEG)
    out_ref[0, 0] = -(pos_term + neg_term) / (np.float32(2.0) * sum_w)


_loss_tc = pl.pallas_call(
    _loss_body,
    out_shape=jax.ShapeDtypeStruct((1, 1), jnp.float32),
    out_specs=pl.BlockSpec(memory_space=pltpu.SMEM),
)


def kernel(entity_embedding, offset_embedding, answer_embedding,
           center_mul, center_add, offset_mul, offset_add,
           subsampling_weight, anchors, relations,
           positive_sample, negative_sample):
    anchors = anchors.astype(jnp.int32)
    relations = relations.astype(jnp.int32)
    positive_sample = positive_sample.astype(jnp.int32)
    neg2 = negative_sample.astype(jnp.int32).reshape(NEGROWS, GROW)

    pos_logit, neg_logit = _sc_logits(
        entity_embedding, offset_embedding, answer_embedding,
        center_mul, center_add, offset_mul, offset_add,
        anchors, relations, positive_sample, neg2)

    loss = _loss_tc(pos_logit.reshape(8, DIM),
                    neg_logit.reshape(BATCH, NNEG),
                    subsampling_weight.reshape(8, DIM),
                    subsampling_weight.reshape(BATCH, 1))
    return loss[0, 0]
